# baseline (speedup 1.0000x reference)
import jax
import jax.numpy as jnp
from jax import lax
from jax.experimental import pallas as pl
from jax.experimental.pallas import tpu as pltpu
from jax.experimental.pallas import tpu_sc as plsc

_DIM = 128
_SEQ = 1000
_ROWS = 64
_LAST_BASE = _SEQ - _ROWS


def _body(x_hbm, wl_hbm, wp_hbm, out_hbm, idx_v, rows_v, pos_v, sem, sem_pos):
    wid = lax.axis_index("s")
    base = lax.min(wid * _ROWS, _LAST_BASE)

    pos_cp = pltpu.async_copy(wp_hbm.at[pl.ds(base, _ROWS)], pos_v, sem_pos)
    pltpu.sync_copy(x_hbm.at[pl.ds(base, _ROWS)], idx_v)
    gather = pltpu.async_copy(wl_hbm.at[idx_v], rows_v, sem)
    pos_cp.wait()
    gather.wait()

    @plsc.parallel_loop(0, _ROWS, unroll=2)
    def _add(r):
        for c in range(_DIM // 16):
            sl = pl.ds(c * 16, 16)
            rows_v[r, sl] = rows_v[r, sl] + pos_v[r, sl]
    pltpu.sync_copy(rows_v, out_hbm.at[pl.ds(base, _ROWS)])


@jax.jit
def kernel(x, W_lettre, W_pos):
    mesh = plsc.VectorSubcoreMesh(core_axis_name="c", subcore_axis_name="s",
                                  num_cores=1)
    f = pl.kernel(
        _body,
        mesh=mesh,
        out_type=jax.ShapeDtypeStruct((_SEQ, _DIM), jnp.float32),
        scratch_types=[
            pltpu.VMEM((_ROWS,), jnp.int32),
            pltpu.VMEM((_ROWS, _DIM), jnp.float32),
            pltpu.VMEM((_ROWS, _DIM), jnp.float32),
            pltpu.SemaphoreType.DMA,
            pltpu.SemaphoreType.DMA,
        ],
    )
    return f(x, W_lettre, W_pos)


# speedup vs baseline: 1.0120x; 1.0120x over previous
import jax
import jax.numpy as jnp
from jax import lax
from jax.experimental import pallas as pl
from jax.experimental.pallas import tpu as pltpu
from jax.experimental.pallas import tpu_sc as plsc

_DIM = 128
_SEQ = 1000
_ROWS = 64
_LAST_BASE = _SEQ - _ROWS


def _body(x_hbm, wl_hbm, wp_hbm, out_hbm, idx_v, rows_v, pos_v, sem, sem_pos):
    wid = lax.axis_index("s")
    base = lax.min(wid * _ROWS, _LAST_BASE)

    pos_cp = pltpu.async_copy(wp_hbm.at[pl.ds(base, _ROWS)], pos_v, sem_pos)
    pltpu.sync_copy(x_hbm.at[pl.ds(base, _ROWS)], idx_v)
    gather = pltpu.async_copy(wl_hbm.at[idx_v], rows_v, sem)
    pos_cp.wait()
    gather.wait()

    def add_row(r, carry):
        for c in range(_DIM // 16):
            sl = pl.ds(c * 16, 16)
            rows_v[r, sl] = rows_v[r, sl] + pos_v[r, sl]
        return carry

    lax.fori_loop(0, _ROWS, add_row, 0)
    pltpu.sync_copy(rows_v, out_hbm.at[pl.ds(base, _ROWS)])


@jax.jit
def kernel(x, W_lettre, W_pos):
    mesh = plsc.VectorSubcoreMesh(core_axis_name="c", subcore_axis_name="s",
                                  num_cores=1)
    f = pl.kernel(
        _body,
        mesh=mesh,
        out_type=jax.ShapeDtypeStruct((_SEQ, _DIM), jnp.float32),
        scratch_types=[
            pltpu.VMEM((_ROWS,), jnp.int32),
            pltpu.VMEM((_ROWS, _DIM), jnp.float32),
            pltpu.VMEM((_ROWS, _DIM), jnp.float32),
            pltpu.SemaphoreType.DMA,
            pltpu.SemaphoreType.DMA,
        ],
    )
    return f(x, W_lettre, W_pos)
